# baseline (speedup 1.0000x reference)
import functools

import jax
import jax.numpy as jnp
from jax import lax
from jax.experimental import pallas as pl
from jax.experimental.pallas import tpu as pltpu
from jax.experimental.pallas import tpu_sc as plsc

N = 10000
E = 320000
D = 128
H = 8
HD = 16

NC = 2
NS = 16
NW = NC * NS
CHUNK = 512
TCH = E // CHUNK
SUB = 128
NSUB = CHUNK // SUB
NCH_LO = TCH // NW
RICH = TCH - NCH_LO * NW
NCH_HI = NCH_LO + 1
RPS = 624
SUBB = 32
NSUBB = CHUNK // SUBB
PERM = (0, 4, 2, 6, 1, 5, 3, 7)
NP = 10240

_f32 = jnp.float32
_i32 = jnp.int32


def _bc(v, e):
    return lax.broadcast(v[e], (16,))


def _allsum(x):
    iota = lax.iota(_i32, 16)
    for s in (1, 2, 4, 8):
        x = x + x[jnp.bitwise_xor(iota, s)]
    return x


def _allmax(x):
    iota = lax.iota(_i32, 16)
    for s in (1, 2, 4, 8):
        x = jnp.maximum(x, x[jnp.bitwise_xor(iota, s)])
    return x


def _tc1_body(x_ref, wq_ref, wk_ref, wv_ref, ws_ref,
              bq_ref, bk_ref, bv_ref, bs_ref,
              q_ref, k_ref, v_ref, skip_ref):
    xb = x_ref[...]
    q_ref[...] = jnp.dot(xb, wq_ref[...], preferred_element_type=_f32) + bq_ref[...]
    k_ref[...] = jnp.dot(xb, wk_ref[...], preferred_element_type=_f32) + bk_ref[...]
    v_ref[...] = jnp.dot(xb, wv_ref[...], preferred_element_type=_f32) + bv_ref[...]
    skip_ref[...] = jnp.dot(xb, ws_ref[...], preferred_element_type=_f32) + bs_ref[...]


def _tc1(x, wqc, wkc, wvc, wsc, bq2, bk2, bv2, bs2):
    blk = 400
    grid = N // blk
    full = lambda s: pl.BlockSpec(s, lambda i: (0, 0))
    row = pl.BlockSpec((blk, D), lambda i: (i, 0))
    return pl.pallas_call(
        _tc1_body,
        grid=(grid,),
        in_specs=[row, full((D, D)), full((D, D)), full((D, D)), full((D, D)),
                  full((1, D)), full((1, D)), full((1, D)), full((1, D))],
        out_specs=[row, row, row, row],
        out_shape=[jax.ShapeDtypeStruct((N, D), _f32)] * 4,
    )(x, wqc, wkc, wvc, wsc, bq2, bk2, bv2, bs2)


_MESH = plsc.VectorSubcoreMesh(core_axis_name="c", subcore_axis_name="s")


def _chunk_sched(wid):
    rich = wid < RICH
    nch = jnp.where(rich, NCH_HI, NCH_LO)
    start = jnp.where(rich, NCH_HI * wid, NCH_LO * wid + RICH)
    return start, nch


@functools.partial(
    pl.kernel,
    mesh=_MESH,
    out_type=(
        jax.ShapeDtypeStruct((16 * E,), _f32),
        jax.ShapeDtypeStruct((NW, 128), _f32),
    ),
    scratch_types=[
        pltpu.VMEM((NSUB, SUB), _i32),
        pltpu.VMEM((NSUB, SUB), _i32),
        pltpu.VMEM((SUB, D), _f32),
        pltpu.VMEM((SUB, D), _f32),
        pltpu.VMEM((SUB, D), _f32),
        pltpu.VMEM((SUB, D), _f32),
        pltpu.VMEM((H, 16), _f32),
        pltpu.VMEM((16 * CHUNK,), _f32),
        pltpu.VMEM((CHUNK,), _f32),
        pltpu.VMEM((128,), _f32),
        pltpu.SemaphoreType.DMA,
        pltpu.SemaphoreType.DMA,
        pltpu.SemaphoreType.DMA,
    ],
)
def _sc_a(q_hbm, k_hbm, src_hbm, dst_hbm, ea_hbm, we_hbm, log_out, gmax_out,
          idxd, idxs, qr0, qr1, kr0, kr1, wev, loge, eab, gmb,
          sem_in, sem_g, sem_o):
    cid = lax.axis_index("c")
    sid = lax.axis_index("s")
    wid = sid * NC + cid
    start, nch = _chunk_sched(wid)
    iota = lax.iota(_i32, 16)
    x8 = jnp.bitwise_xor(iota, 8)
    x4 = jnp.bitwise_xor(iota, 4)
    x2 = jnp.bitwise_xor(iota, 2)
    x1 = jnp.bitwise_xor(iota, 1)
    mk8 = iota < 8
    mk4 = (iota & 7) < 4
    mk2 = (iota & 3) < 2
    pltpu.sync_copy(we_hbm, wev)
    for c in range(8):
        gmb[pl.ds(c * 16, 16)] = jnp.zeros((16,), _f32)
    qbuf = (qr0, qr1)
    kbuf = (kr0, kr1)

    def chunk_body(i, mvec):
        base = pl.multiple_of((start + i) * CHUNK, CHUNK)

        @pl.when(i > 0)
        def _():
            pltpu.make_async_copy(loge, log_out.at[pl.ds(0, 16 * CHUNK)],
                                  sem_o).wait()

        ind = []
        for j in range(NSUB):
            ind.append(pltpu.async_copy(
                dst_hbm.at[pl.ds(base + j * SUB, SUB)], idxd.at[j], sem_in))
            ind.append(pltpu.async_copy(
                src_hbm.at[pl.ds(base + j * SUB, SUB)], idxs.at[j], sem_in))
        ind.append(pltpu.async_copy(ea_hbm.at[pl.ds(base, CHUNK)], eab, sem_in))
        for d in ind:
            d.wait()

        pend = None
        for j in range(NSUB):
            if j == 0:
                pend = (pltpu.async_copy(q_hbm.at[idxd.at[0]], qbuf[0], sem_g),
                        pltpu.async_copy(k_hbm.at[idxs.at[0]], kbuf[0], sem_g))
            pend[0].wait()
            pend[1].wait()
            qr = qbuf[j % 2]
            kr = kbuf[j % 2]
            if j + 1 < NSUB:
                nb = (j + 1) % 2
                pend = (pltpu.async_copy(q_hbm.at[idxd.at[j + 1]], qbuf[nb],
                                         sem_g),
                        pltpu.async_copy(k_hbm.at[idxs.at[j + 1]], kbuf[nb],
                                         sem_g))
            wevs = [wev[h] for h in range(H)]

            def grp(g, mv):
                eav = eab[pl.ds(j * SUB + g * 16, 16)]
                for e in range(16):
                    er = g * 16 + e
                    ea_b = _bc(eav, e)
                    prods = []
                    for h in PERM:
                        qv = qr[er, pl.ds(h * HD, HD)]
                        kv = kr[er, pl.ds(h * HD, HD)] + ea_b * wevs[h]
                        prods.append(qv * kv)
                    t = [p + p[x8] for p in prods]
                    m = [jnp.where(mk8, t[2 * i], t[2 * i + 1])
                         for i in range(4)]
                    u = [x + x[x4] for x in m]
                    n = [jnp.where(mk4, u[0], u[1]),
                         jnp.where(mk4, u[2], u[3])]
                    v = [x + x[x2] for x in n]
                    w = jnp.where(mk2, v[0], v[1])
                    f = (w + w[x1]) * 0.25
                    loge[pl.ds((j * SUB + er) * 16, 16)] = f
                    mv = jnp.maximum(mv, f)
                return mv

            mvec = lax.fori_loop(0, SUB // 16, grp, mvec)
        pltpu.async_copy(loge, log_out.at[pl.ds(base * 16, 16 * CHUNK)], sem_o)
        return mvec

    mvec = lax.fori_loop(0, nch, chunk_body, jnp.full((16,), -jnp.inf, _f32))
    pltpu.make_async_copy(loge, log_out.at[pl.ds(0, 16 * CHUNK)], sem_o).wait()
    gmb[pl.ds(0, 16)] = mvec
    pltpu.sync_copy(gmb, gmax_out.at[wid])


@functools.partial(
    pl.kernel,
    mesh=_MESH,
    out_type=(
        jax.ShapeDtypeStruct((NC, N, D), _f32),
        jax.ShapeDtypeStruct((NC, NP * 16), _f32),
    ),
    scratch_types=[
        pltpu.VMEM((NSUBB, SUBB), _i32),
        pltpu.VMEM((NSUBB, SUBB), _i32),
        pltpu.VMEM((SUBB, D), _f32),
        pltpu.VMEM((SUBB, D), _f32),
        pltpu.VMEM((SUBB, D), _f32),
        pltpu.VMEM((SUBB * 16,), _f32),
        pltpu.VMEM((SUBB * 16 // 128, 128), _i32),
        pltpu.VMEM((16 * CHUNK,), _f32),
        pltpu.VMEM((CHUNK,), _f32),
        pltpu.VMEM((128,), _f32),
        pltpu.VMEM_SHARED((N, D), _f32),
        pltpu.VMEM_SHARED((NP * 16,), _f32),
        pltpu.SemaphoreType.DMA,
        pltpu.SemaphoreType.DMA,
        pltpu.SemaphoreType.DMA,
    ],
)
def _sc_b(v_hbm, src_hbm, dst_hbm, ea_hbm, log_hbm, gmax_hbm,
          vs_out, ds_out,
          idxd, idxs, vr0, vr1, ctr, ctr2, idxf, logb, eab, gmx,
          vs_sp, ds_sp, sem_in, sem_g, sem_s):
    cid = lax.axis_index("c")
    sid = lax.axis_index("s")
    wid = sid * NC + cid
    start, nch = _chunk_sched(wid)
    iota = lax.iota(_i32, 16)
    odd = (iota & 1) == 1
    one16 = jnp.ones((16,), _f32)
    vbuf = (vr0, vr1)

    gredv = jnp.full((16,), -jnp.inf, _f32)
    for w in range(NW):
        pltpu.sync_copy(gmax_hbm.at[w], gmx)
        gredv = jnp.maximum(gredv, gmx[pl.ds(0, 16)])

    def zero_body(r, carry):
        for c in range(D // 16):
            ctr[r, pl.ds(c * 16, 16)] = jnp.zeros((16,), _f32)
        ctr2[pl.ds(r * 16, 16)] = jnp.zeros((16,), _f32)
        return carry

    lax.fori_loop(0, SUBB, zero_body, 0)
    r0 = sid * RPS
    for p in range(RPS // SUBB):
        pltpu.sync_copy(ctr, vs_sp.at[pl.ds(r0 + p * SUBB, SUBB)])
    pltpu.sync_copy(ctr.at[pl.ds(0, RPS % SUBB)],
                    vs_sp.at[pl.ds(r0 + (RPS // SUBB) * SUBB, RPS % SUBB)])

    @pl.when(sid == NS - 1)
    def _():
        pltpu.sync_copy(ctr.at[pl.ds(0, N - NS * RPS)],
                        vs_sp.at[pl.ds(NS * RPS, N - NS * RPS)])

    rd0 = sid * (NP * 16 // NS)
    for p in range(NP * 16 // NS // (SUBB * 16)):
        pltpu.sync_copy(ctr2, ds_sp.at[pl.ds(rd0 + p * SUBB * 16, SUBB * 16)])

    plsc.subcore_barrier()

    def _drain_scatters():
        pltpu.make_async_copy(ctr, vs_sp.at[idxd.at[NSUBB - 1]], sem_s).wait()
        for t in range(SUBB * 16 // 128):
            pltpu.make_async_copy(ctr2.at[pl.ds(t * 128, 128)],
                                  ds_sp.at[idxf.at[t]], sem_s).wait()

    def chunk_body(i, carry):
        base = pl.multiple_of((start + i) * CHUNK, CHUNK)

        @pl.when(i > 0)
        def _():
            _drain_scatters()

        ind = []
        for j in range(NSUBB):
            ind.append(pltpu.async_copy(
                dst_hbm.at[pl.ds(base + j * SUBB, SUBB)], idxd.at[j], sem_in))
            ind.append(pltpu.async_copy(
                src_hbm.at[pl.ds(base + j * SUBB, SUBB)], idxs.at[j], sem_in))
        ind.append(pltpu.async_copy(ea_hbm.at[pl.ds(base, CHUNK)], eab, sem_in))
        ind.append(pltpu.async_copy(log_hbm.at[pl.ds(base * 16, 16 * CHUNK)],
                                    logb, sem_in))
        for d in ind:
            d.wait()

        pend = None
        sc_pend = None
        for j in range(NSUBB):
            if j == 0:
                pend = pltpu.async_copy(v_hbm.at[idxs.at[0]], vbuf[0], sem_g)
            pend.wait()
            vr = vbuf[j % 2]
            if j + 1 < NSUBB:
                pend = pltpu.async_copy(v_hbm.at[idxs.at[j + 1]],
                                        vbuf[(j + 1) % 2], sem_g)
            if sc_pend is not None:
                for d in sc_pend:
                    d.wait()

            def grp(g, carry2):
                eav = eab[pl.ds(j * SUBB + g * 16, 16)]
                dstv = idxd[j, pl.ds(g * 16, 16)]
                for e in range(16):
                    er = g * 16 + e
                    lg = logb[pl.ds((j * SUBB + er) * 16, 16)]
                    ex = jnp.exp(lg - gredv)
                    ea_b = _bc(eav, e)
                    ctr2[pl.ds(er * 16, 16)] = ex * jnp.where(odd, ea_b, one16)
                    idxf[er // 8, pl.ds((er % 8) * 16, 16)] = (
                        _bc(dstv, e).astype(_i32) * 16 + iota)
                    for h in range(H):
                        ex_b = _bc(ex, 2 * h)
                        ctr[er, pl.ds(h * HD, HD)] = (
                            vr[er, pl.ds(h * HD, HD)] * ex_b)
                return carry2

            lax.fori_loop(0, SUBB // 16, grp, 0)
            sc_pend = [pltpu.async_copy(ctr, vs_sp.at[idxd.at[j]], sem_s,
                                        add=True)]
            for t in range(SUBB * 16 // 128):
                sc_pend.append(pltpu.async_copy(
                    ctr2.at[pl.ds(t * 128, 128)], ds_sp.at[idxf.at[t]],
                    sem_s, add=True))
        return carry

    lax.fori_loop(0, nch, chunk_body, 0)
    _drain_scatters()
    plsc.subcore_barrier()
    pltpu.sync_copy(vs_sp.at[pl.ds(r0, RPS)], vs_out.at[cid].at[pl.ds(r0, RPS)])

    @pl.when(sid == NS - 1)
    def _():
        pltpu.sync_copy(vs_sp.at[pl.ds(NS * RPS, N - NS * RPS)],
                        vs_out.at[cid].at[pl.ds(NS * RPS, N - NS * RPS)])

    pltpu.sync_copy(ds_sp.at[pl.ds(rd0, NP * 16 // NS)],
                    ds_out.at[cid].at[pl.ds(rd0, NP * 16 // NS)])


def _tc2_body(vs0_ref, vs1_ref, ds0_ref, ds1_ref, skip_ref, x_ref,
              repd_ref, repswe_ref, wo_ref, bo_ref, g_ref, b_ref, out_ref):
    vsum = vs0_ref[...] + vs1_ref[...]
    dss = ds0_ref[...] + ds1_ref[...]
    den128 = jnp.dot(dss, repd_ref[...], preferred_element_type=_f32)
    den128 = jnp.where(den128 == 0.0, 1.0, den128)
    swe128 = jnp.dot(dss, repswe_ref[...], preferred_element_type=_f32)
    mh = (vsum + swe128) / den128 + skip_ref[...]
    out = (jnp.dot(mh, wo_ref[...], preferred_element_type=_f32)
           + bo_ref[...] + x_ref[...])
    mu = jnp.mean(out, axis=1, keepdims=True)
    var = jnp.mean((out - mu) * (out - mu), axis=1, keepdims=True)
    out_ref[...] = (out - mu) * lax.rsqrt(var + 1e-5) * g_ref[...] + b_ref[...]


def _tc2(vs0, vs1, ds0, ds1, skip, x, repd, repswe, wo, bo2, g2, b2):
    blk = 512
    grid = pl.cdiv(N, blk)
    full = lambda s: pl.BlockSpec(s, lambda i: (0, 0))
    row = pl.BlockSpec((blk, D), lambda i: (i, 0))
    row16 = pl.BlockSpec((blk, 16), lambda i: (i, 0))
    return pl.pallas_call(
        _tc2_body,
        grid=(grid,),
        in_specs=[row, row, row16, row16, row, row,
                  full((16, D)), full((16, D)), full((D, D)),
                  full((1, D)), full((1, D)), full((1, D))],
        out_specs=row,
        out_shape=jax.ShapeDtypeStruct((N, D), _f32),
    )(vs0, vs1, ds0, ds1, skip, x, repd, repswe, wo, bo2, g2, b2)


def kernel(x, edge_index, edge_attr, Wq, bq, Wk, bk, Wv, bv, We, Wskip, bskip,
           Wo, bo, gamma, beta):
    src = edge_index[0]
    dst = edge_index[1]
    wqc = jnp.transpose(Wq, (1, 0, 2)).reshape(D, D)
    wkc = jnp.transpose(Wk, (1, 0, 2)).reshape(D, D)
    wvc = jnp.transpose(Wv, (1, 0, 2)).reshape(D, D)
    wsc = jnp.transpose(Wskip, (1, 0, 2)).reshape(D, D)
    we_row = We.reshape(1, D)
    we_hd = We.reshape(H, HD)
    base = jnp.repeat(jnp.eye(H, dtype=_f32), HD, axis=1)
    repd = jnp.zeros((16, D), _f32).at[0::2, :].set(base)
    repswe = jnp.zeros((16, D), _f32).at[1::2, :].set(base * we_row)

    q, k, v, skip = _tc1(x, wqc, wkc, wvc, wsc,
                         bq.reshape(1, D), bk.reshape(1, D),
                         bv.reshape(1, D), bskip.reshape(1, D))
    log_e, gmax = _sc_a(q, k, src, dst, edge_attr, we_hd)
    vs, ds = _sc_b(v, src, dst, edge_attr, log_e, gmax)
    ds0 = ds[0].reshape(NP, 16)
    ds1 = ds[1].reshape(NP, 16)
    out = _tc2(vs[0], vs[1], ds0, ds1, skip, x, repd, repswe,
               Wo, bo.reshape(1, D), gamma.reshape(1, D), beta.reshape(1, D))
    return out


# speedup vs baseline: 1.4070x; 1.4070x over previous
import functools

import jax
import jax.numpy as jnp
from jax import lax
from jax.experimental import pallas as pl
from jax.experimental.pallas import tpu as pltpu
from jax.experimental.pallas import tpu_sc as plsc

N = 10000
E = 320000
D = 128
H = 8
HD = 16

NC = 2
NS = 16
NW = NC * NS
CHUNK = 512
TCH = E // CHUNK
SUB = 128
NSUB = CHUNK // SUB
NCH_LO = TCH // NW
RICH = TCH - NCH_LO * NW
NCH_HI = NCH_LO + 1
RPS = 624
SUBB = 64
NSUBB = CHUNK // SUBB
NP = 10240

_f32 = jnp.float32
_i32 = jnp.int32


def _bc(v, e):
    return lax.broadcast(v[e], (16,))


def _allsum(x):
    iota = lax.iota(_i32, 16)
    for s in (1, 2, 4, 8):
        x = x + x[jnp.bitwise_xor(iota, s)]
    return x


def _allmax(x):
    iota = lax.iota(_i32, 16)
    for s in (1, 2, 4, 8):
        x = jnp.maximum(x, x[jnp.bitwise_xor(iota, s)])
    return x


def _tc1_body(x_ref, wq_ref, wk_ref, wv_ref, ws_ref,
              bq_ref, bk_ref, bv_ref, bs_ref,
              q_ref, k_ref, v_ref, skip_ref):
    xb = x_ref[...]
    q_ref[...] = jnp.dot(xb, wq_ref[...], preferred_element_type=_f32) + bq_ref[...]
    k_ref[...] = jnp.dot(xb, wk_ref[...], preferred_element_type=_f32) + bk_ref[...]
    v_ref[...] = jnp.dot(xb, wv_ref[...], preferred_element_type=_f32) + bv_ref[...]
    skip_ref[...] = jnp.dot(xb, ws_ref[...], preferred_element_type=_f32) + bs_ref[...]


def _tc1(x, wqc, wkc, wvc, wsc, bq2, bk2, bv2, bs2):
    blk = 400
    grid = N // blk
    full = lambda s: pl.BlockSpec(s, lambda i: (0, 0))
    row = pl.BlockSpec((blk, D), lambda i: (i, 0))
    return pl.pallas_call(
        _tc1_body,
        grid=(grid,),
        in_specs=[row, full((D, D)), full((D, D)), full((D, D)), full((D, D)),
                  full((1, D)), full((1, D)), full((1, D)), full((1, D))],
        out_specs=[row, row, row, row],
        out_shape=[jax.ShapeDtypeStruct((N, D), _f32)] * 4,
    )(x, wqc, wkc, wvc, wsc, bq2, bk2, bv2, bs2)


_MESH = plsc.VectorSubcoreMesh(core_axis_name="c", subcore_axis_name="s")


def _chunk_sched(wid):
    rich = wid < RICH
    nch = jnp.where(rich, NCH_HI, NCH_LO)
    start = jnp.where(rich, NCH_HI * wid, NCH_LO * wid + RICH)
    return start, nch


@functools.partial(
    pl.kernel,
    mesh=_MESH,
    out_type=(
        jax.ShapeDtypeStruct((H * E,), _f32),
        jax.ShapeDtypeStruct((NW, H * 16), _f32),
    ),
    scratch_types=[
        pltpu.VMEM((NSUB, SUB), _i32),
        pltpu.VMEM((NSUB, SUB), _i32),
        pltpu.VMEM((SUB, D), _f32),
        pltpu.VMEM((SUB, D), _f32),
        pltpu.VMEM((SUB, D), _f32),
        pltpu.VMEM((SUB, D), _f32),
        pltpu.VMEM((H, 16), _f32),
        pltpu.VMEM((H * CHUNK,), _f32),
        pltpu.VMEM((CHUNK,), _f32),
        pltpu.VMEM((H * 16,), _f32),
        pltpu.SemaphoreType.DMA,
        pltpu.SemaphoreType.DMA,
        pltpu.SemaphoreType.DMA,
    ],
)
def _sc_a(q_hbm, k_hbm, src_hbm, dst_hbm, ea_hbm, we_hbm, log_out, gmax_out,
          idxd, idxs, qr0, qr1, kr0, kr1, wev, logb, eab, gmb,
          sem_in, sem_g, sem_o):
    cid = lax.axis_index("c")
    sid = lax.axis_index("s")
    wid = sid * NC + cid
    start, nch = _chunk_sched(wid)
    iota = lax.iota(_i32, 16)
    pltpu.sync_copy(we_hbm, wev)
    for h in range(H):
        gmb[pl.ds(h * 16, 16)] = jnp.full((16,), -jnp.inf, _f32)
    qbuf = (qr0, qr1)
    kbuf = (kr0, kr1)

    def chunk_body(i, carry):
        base = pl.multiple_of((start + i) * CHUNK, CHUNK)

        @pl.when(i > 0)
        def _():
            for h in range(H):
                pltpu.make_async_copy(
                    logb.at[pl.ds(h * CHUNK, CHUNK)],
                    log_out.at[pl.ds(h * E + base, CHUNK)], sem_o).wait()

        ind = []
        for j in range(NSUB):
            ind.append(pltpu.async_copy(
                dst_hbm.at[pl.ds(base + j * SUB, SUB)], idxd.at[j], sem_in))
            ind.append(pltpu.async_copy(
                src_hbm.at[pl.ds(base + j * SUB, SUB)], idxs.at[j], sem_in))
        ind.append(pltpu.async_copy(ea_hbm.at[pl.ds(base, CHUNK)], eab, sem_in))
        for d in ind:
            d.wait()

        pend = None
        for j in range(NSUB):
            if j == 0:
                pend = (pltpu.async_copy(q_hbm.at[idxd.at[0]], qbuf[0], sem_g),
                        pltpu.async_copy(k_hbm.at[idxs.at[0]], kbuf[0], sem_g))
            pend[0].wait()
            pend[1].wait()
            qr = qbuf[j % 2]
            kr = kbuf[j % 2]
            if j + 1 < NSUB:
                nb = (j + 1) % 2
                pend = (pltpu.async_copy(q_hbm.at[idxd.at[j + 1]], qbuf[nb],
                                         sem_g),
                        pltpu.async_copy(k_hbm.at[idxs.at[j + 1]], kbuf[nb],
                                         sem_g))
            wevs = [wev[h] for h in range(H)]

            def grp(g, carry2):
                eav = eab[pl.ds(j * SUB + g * 16, 16)]
                accs = [jnp.zeros((16,), _f32) for _ in range(H)]
                for e in range(16):
                    er = g * 16 + e
                    ea_b = _bc(eav, e)
                    lane = iota == e
                    for h in range(H):
                        qv = qr[er, pl.ds(h * HD, HD)]
                        kv = kr[er, pl.ds(h * HD, HD)] + ea_b * wevs[h]
                        lg = _allsum(qv * kv)
                        accs[h] = jnp.where(lane, lg, accs[h])
                for h in range(H):
                    lgv = accs[h] * 0.25
                    logb[pl.ds(h * CHUNK + j * SUB + g * 16, 16)] = lgv
                    gmb[pl.ds(h * 16, 16)] = jnp.maximum(
                        gmb[pl.ds(h * 16, 16)], lgv)
                return carry2

            lax.fori_loop(0, SUB // 16, grp, 0)
        for h in range(H):
            pltpu.async_copy(logb.at[pl.ds(h * CHUNK, CHUNK)],
                             log_out.at[pl.ds(h * E + base, CHUNK)], sem_o)
        return carry

    lax.fori_loop(0, nch, chunk_body, 0)
    for h in range(H):
        pltpu.make_async_copy(logb.at[pl.ds(h * CHUNK, CHUNK)],
                              log_out.at[pl.ds(h * E, CHUNK)], sem_o).wait()
    pltpu.sync_copy(gmb, gmax_out.at[wid])


@functools.partial(
    pl.kernel,
    mesh=_MESH,
    out_type=(
        jax.ShapeDtypeStruct((NC, N, D), _f32),
        jax.ShapeDtypeStruct((NC, 2 * H, NP), _f32),
    ),
    scratch_types=[
        pltpu.VMEM((NSUBB, SUBB), _i32),
        pltpu.VMEM((NSUBB, SUBB), _i32),
        pltpu.VMEM((2 * H, SUBB), _i32),
        pltpu.VMEM((SUBB, D), _f32),
        pltpu.VMEM((SUBB, D), _f32),
        pltpu.VMEM((SUBB, D), _f32),
        pltpu.VMEM((2 * H * SUBB,), _f32),
        pltpu.VMEM((H * CHUNK,), _f32),
        pltpu.VMEM((CHUNK,), _f32),
        pltpu.VMEM((H * 16,), _f32),
        pltpu.VMEM((H, 16), _f32),
        pltpu.VMEM_SHARED((N, D), _f32),
        pltpu.VMEM_SHARED((2 * H * NP,), _f32),
        pltpu.SemaphoreType.DMA,
        pltpu.SemaphoreType.DMA,
        pltpu.SemaphoreType.DMA,
    ],
)
def _sc_b(v_hbm, src_hbm, dst_hbm, ea_hbm, log_hbm, gmax_hbm,
          vs_out, ds_out,
          idxd, idxs, idxf, vr0, vr1, ctr, exb, logb, eab, gmx, gred,
          vs_sp, ds_sp, sem_in, sem_g, sem_s):
    cid = lax.axis_index("c")
    sid = lax.axis_index("s")
    wid = sid * NC + cid
    start, nch = _chunk_sched(wid)
    vbuf = (vr0, vr1)

    for h in range(H):
        gred[h] = jnp.full((16,), -jnp.inf, _f32)
    for w in range(NW):
        pltpu.sync_copy(gmax_hbm.at[w], gmx)
        for h in range(H):
            gred[h] = jnp.maximum(gred[h], gmx[pl.ds(h * 16, 16)])
    for h in range(H):
        gred[h] = _allmax(gred[h])

    def zero_body(r, carry):
        for c in range(D // 16):
            ctr[r, pl.ds(c * 16, 16)] = jnp.zeros((16,), _f32)
        return carry

    lax.fori_loop(0, SUBB, zero_body, 0)

    def zero_exb(t, carry):
        exb[pl.ds(t * 16, 16)] = jnp.zeros((16,), _f32)
        return carry

    lax.fori_loop(0, 2 * H * SUBB // 16, zero_exb, 0)

    r0 = sid * RPS
    for p in range(RPS // SUBB):
        pltpu.sync_copy(ctr, vs_sp.at[pl.ds(r0 + p * SUBB, SUBB)])
    pltpu.sync_copy(ctr.at[pl.ds(0, RPS % SUBB)],
                    vs_sp.at[pl.ds(r0 + (RPS // SUBB) * SUBB, RPS % SUBB)])

    @pl.when(sid == NS - 1)
    def _():
        pltpu.sync_copy(ctr.at[pl.ds(0, N - NS * RPS)],
                        vs_sp.at[pl.ds(NS * RPS, N - NS * RPS)])

    zlen = 2 * H * SUBB
    for p in range(NP // zlen):
        pltpu.sync_copy(exb, ds_sp.at[pl.ds(sid * NP + p * zlen, zlen)])

    plsc.subcore_barrier()

    def _drain_scatters(last_j):
        pltpu.make_async_copy(ctr, vs_sp.at[idxd.at[last_j]], sem_s).wait()
        for t in range(2 * H):
            pltpu.make_async_copy(exb.at[pl.ds(t * SUBB, SUBB)],
                                  ds_sp.at[idxf.at[t]], sem_s).wait()

    def chunk_body(i, carry):
        base = pl.multiple_of((start + i) * CHUNK, CHUNK)

        @pl.when(i > 0)
        def _():
            _drain_scatters(NSUBB - 1)

        ind = []
        for j in range(NSUBB):
            ind.append(pltpu.async_copy(
                dst_hbm.at[pl.ds(base + j * SUBB, SUBB)], idxd.at[j], sem_in))
            ind.append(pltpu.async_copy(
                src_hbm.at[pl.ds(base + j * SUBB, SUBB)], idxs.at[j], sem_in))
        ind.append(pltpu.async_copy(ea_hbm.at[pl.ds(base, CHUNK)], eab, sem_in))
        for h in range(H):
            ind.append(pltpu.async_copy(
                log_hbm.at[pl.ds(h * E + base, CHUNK)],
                logb.at[pl.ds(h * CHUNK, CHUNK)], sem_in))
        for d in ind:
            d.wait()

        pend = None
        sc_pend = None
        for j in range(NSUBB):
            if j == 0:
                pend = pltpu.async_copy(v_hbm.at[idxs.at[0]], vbuf[0], sem_g)
            pend.wait()
            vr = vbuf[j % 2]
            if j + 1 < NSUBB:
                pend = pltpu.async_copy(v_hbm.at[idxs.at[j + 1]],
                                        vbuf[(j + 1) % 2], sem_g)
            if sc_pend is not None:
                for d in sc_pend:
                    d.wait()

            def grp(g, carry2):
                eav = eab[pl.ds(j * SUBB + g * 16, 16)]
                dstv = idxd[j, pl.ds(g * 16, 16)]
                exs = []
                for h in range(H):
                    lg = logb[pl.ds(h * CHUNK + j * SUBB + g * 16, 16)]
                    ex = jnp.exp(lg - gred[h])
                    exb[pl.ds(h * SUBB + g * 16, 16)] = ex
                    exb[pl.ds((H + h) * SUBB + g * 16, 16)] = ex * eav
                    idxf[h, pl.ds(g * 16, 16)] = dstv + h * NP
                    idxf[H + h, pl.ds(g * 16, 16)] = dstv + (H + h) * NP
                    exs.append(ex)
                for e in range(16):
                    er = g * 16 + e
                    for h in range(H):
                        ex_b = _bc(exs[h], e)
                        ctr[er, pl.ds(h * HD, HD)] = (
                            vr[er, pl.ds(h * HD, HD)] * ex_b)
                return carry2

            lax.fori_loop(0, SUBB // 16, grp, 0)
            sc_pend = [pltpu.async_copy(ctr, vs_sp.at[idxd.at[j]], sem_s,
                                        add=True)]
            for t in range(2 * H):
                sc_pend.append(pltpu.async_copy(
                    exb.at[pl.ds(t * SUBB, SUBB)], ds_sp.at[idxf.at[t]],
                    sem_s, add=True))
        return carry

    lax.fori_loop(0, nch, chunk_body, 0)
    _drain_scatters(NSUBB - 1)
    plsc.subcore_barrier()
    pltpu.sync_copy(vs_sp.at[pl.ds(r0, RPS)], vs_out.at[cid].at[pl.ds(r0, RPS)])

    @pl.when(sid == NS - 1)
    def _():
        pltpu.sync_copy(vs_sp.at[pl.ds(NS * RPS, N - NS * RPS)],
                        vs_out.at[cid].at[pl.ds(NS * RPS, N - NS * RPS)])

    pltpu.sync_copy(ds_sp.at[pl.ds(sid * NP, NP)], ds_out.at[cid].at[sid])


def _tc2_body(vs0_ref, vs1_ref, ds0_ref, ds1_ref, skip_ref, x_ref,
              we_ref, rep_ref, wo_ref, bo_ref, g_ref, b_ref, out_ref):
    vsum = vs0_ref[...] + vs1_ref[...]
    dss = ds0_ref[...] + ds1_ref[...]
    den = dss[:H, :]
    s = dss[H:, :]
    den = jnp.where(den == 0.0, 1.0, den)
    dn = (((0,), (0,)), ((), ()))
    den128 = lax.dot_general(den, rep_ref[...], dn,
                             preferred_element_type=_f32)
    s128 = lax.dot_general(s, rep_ref[...], dn, preferred_element_type=_f32)
    mh = (vsum + s128 * we_ref[...]) / den128 + skip_ref[...]
    out = (jnp.dot(mh, wo_ref[...], preferred_element_type=_f32)
           + bo_ref[...] + x_ref[...])
    mu = jnp.mean(out, axis=1, keepdims=True)
    var = jnp.mean((out - mu) * (out - mu), axis=1, keepdims=True)
    out_ref[...] = (out - mu) * lax.rsqrt(var + 1e-5) * g_ref[...] + b_ref[...]


def _tc2(vs0, vs1, ds0, ds1, skip, x, we_row, rep, wo, bo2, g2, b2):
    blk = 512
    grid = pl.cdiv(N, blk)
    full = lambda s: pl.BlockSpec(s, lambda i: (0, 0))
    row = pl.BlockSpec((blk, D), lambda i: (i, 0))
    colb = pl.BlockSpec((2 * H, blk), lambda i: (0, i))
    return pl.pallas_call(
        _tc2_body,
        grid=(grid,),
        in_specs=[row, row, colb, colb, row, row,
                  full((1, D)), full((H, D)), full((D, D)),
                  full((1, D)), full((1, D)), full((1, D))],
        out_specs=row,
        out_shape=jax.ShapeDtypeStruct((N, D), _f32),
    )(vs0, vs1, ds0, ds1, skip, x, we_row, rep, wo, bo2, g2, b2)


def kernel(x, edge_index, edge_attr, Wq, bq, Wk, bk, Wv, bv, We, Wskip, bskip,
           Wo, bo, gamma, beta):
    src = edge_index[0]
    dst = edge_index[1]
    wqc = jnp.transpose(Wq, (1, 0, 2)).reshape(D, D)
    wkc = jnp.transpose(Wk, (1, 0, 2)).reshape(D, D)
    wvc = jnp.transpose(Wv, (1, 0, 2)).reshape(D, D)
    wsc = jnp.transpose(Wskip, (1, 0, 2)).reshape(D, D)
    we_row = We.reshape(1, D)
    we_hd = We.reshape(H, HD)
    rep = jnp.repeat(jnp.eye(H, dtype=_f32), HD, axis=1)

    q, k, v, skip = _tc1(x, wqc, wkc, wvc, wsc,
                         bq.reshape(1, D), bk.reshape(1, D),
                         bv.reshape(1, D), bskip.reshape(1, D))
    log_e, gmax = _sc_a(q, k, src, dst, edge_attr, we_hd)
    vs, ds = _sc_b(v, src, dst, edge_attr, log_e, gmax)
    out = _tc2(vs[0], vs[1], ds[0], ds[1], skip, x, we_row, rep,
               Wo, bo.reshape(1, D), gamma.reshape(1, D), beta.reshape(1, D))
    return out
